# baseline (speedup 1.0000x reference)
import functools

import jax
import jax.numpy as jnp
from jax import lax
from jax.experimental import pallas as pl
from jax.experimental.pallas import tpu as pltpu
from jax.experimental.pallas import tpu_sc as plsc

B, N, C = 4, 32768, 64
M = B * N
NC, NS = 2, 16
NW = NC * NS
RPW = M // NW
F = 128
CHUNKS = RPW // F
BN = 2048
NJ = N // BN


def _make_sc_gather_max():
    mesh = plsc.VectorSubcoreMesh(core_axis_name="c", subcore_axis_name="s")

    row_buf = pltpu.VMEM((F, C), jnp.float32)

    @functools.partial(
        pl.kernel,
        mesh=mesh,
        out_type=jax.ShapeDtypeStruct((M, C), jnp.float32),
        compiler_params=pltpu.CompilerParams(use_tc_tiling_on_sc=False),
        scratch_types=[
            pltpu.VMEM((RPW,), jnp.int32),
            pltpu.VMEM((RPW,), jnp.int32),
            pltpu.VMEM((RPW,), jnp.int32),
            row_buf, row_buf, row_buf, row_buf,
            row_buf, row_buf, row_buf, row_buf,
            pltpu.SemaphoreType.DMA,
            pltpu.SemaphoreType.DMA,
            pltpu.SemaphoreType.DMA,
        ],
    )
    def sc_gather_max(table_hbm, nbr0_hbm, nbr1_hbm, nbr2_hbm, out_hbm,
                      idx0, idx1, idx2,
                      g0a, g1a, g2a, acca,
                      g0b, g1b, g2b, accb,
                      sema, semb, wsem):
        wid = lax.axis_index("s") * NC + lax.axis_index("c")
        wbase = wid * RPW
        bufs = ((g0a, g1a, g2a, acca, sema), (g0b, g1b, g2b, accb, semb))
        idxs = (idx0, idx1, idx2)

        pltpu.sync_copy(nbr0_hbm.at[pl.ds(wbase, RPW)], idx0)
        pltpu.sync_copy(nbr1_hbm.at[pl.ds(wbase, RPW)], idx1)
        pltpu.sync_copy(nbr2_hbm.at[pl.ds(wbase, RPW)], idx2)

        def issue(g, s):
            g0, g1, g2, acc, sem = bufs[s]
            sl = pl.ds(g * F, F)
            pltpu.async_copy(table_hbm.at[idx0.at[sl]], g0, sem)
            pltpu.async_copy(table_hbm.at[idx1.at[sl]], g1, sem)
            pltpu.async_copy(table_hbm.at[idx2.at[sl]], g2, sem)
            pltpu.async_copy(table_hbm.at[pl.ds(wbase + g * F, F)], acc, sem)

        def drain(s):
            g0, g1, g2, acc, sem = bufs[s]
            for dst in (g0, g1, g2, acc):
                pltpu.make_async_copy(table_hbm.at[pl.ds(0, F)], dst, sem).wait()

        def compute(s):
            g0, g1, g2, acc, _ = bufs[s]

            def row_body(r, rc):
                for c in range(C // 16):
                    sl = pl.ds(c * 16, 16)
                    m01 = jnp.maximum(g0[r, sl], g1[r, sl])
                    m23 = jnp.maximum(g2[r, sl], acc[r, sl])
                    acc[r, sl] = jnp.maximum(m01, m23)
                return rc

            lax.fori_loop(0, F, row_body, 0, unroll=2)

        def writeback(g, s):
            acc = bufs[s][3]
            pltpu.async_copy(acc, out_hbm.at[pl.ds(wbase + g * F, F)], wsem)

        def wb_wait():
            pltpu.make_async_copy(acca, out_hbm.at[pl.ds(0, F)], wsem).wait()

        issue(0, 0)

        def pair_body(p, carry):
            for s in range(2):
                g = 2 * p + s

                @pl.when(g >= 1)
                def _():
                    wb_wait()

                @pl.when(g + 1 < CHUNKS)
                def _():
                    issue(g + 1, 1 - s)

                drain(s)
                compute(s)
                writeback(g, s)
            return carry

        lax.fori_loop(0, CHUNKS // 2, pair_body, 0)
        wb_wait()

    return sc_gather_max


_SC_CACHE = []


def _sc_gather_max(table, nbr0, nbr1, nbr2):
    if not _SC_CACHE:
        _SC_CACHE.append(_make_sc_gather_max())
    return _SC_CACHE[0](table, nbr0, nbr1, nbr2)


def _bn_fold(gram, sums, w, gamma, beta):
    inv_m = 1.0 / M
    mean_x = sums * inv_m
    mean_y = jnp.sum(w * mean_x[None, :], axis=1)
    a = jnp.dot(w, gram, preferred_element_type=jnp.float32)
    ey2 = jnp.sum(a * w, axis=1) * inv_m
    var = ey2 - mean_y * mean_y
    scale = gamma * lax.rsqrt(var + 1e-5)
    shift = beta - scale * mean_y
    return jnp.stack([scale, shift])


def _tc_stats1_body(sp_ref, st_ref, w_ref, gb_ref, scsh_ref, gacc, sacc):
    b = pl.program_id(0)
    j = pl.program_id(1)
    x = jnp.concatenate([sp_ref[0], st_ref[0]], axis=0)

    @pl.when((b == 0) & (j == 0))
    def _():
        gacc[...] = jnp.zeros_like(gacc)
        sacc[...] = jnp.zeros_like(sacc)

    gacc[...] += lax.dot_general(x, x, dimension_numbers=(((1,), (1,)), ((), ())),
                                 preferred_element_type=jnp.float32)
    sacc[0, :] += jnp.sum(x, axis=1)

    @pl.when((b == B - 1) & (j == NJ - 1))
    def _():
        scsh_ref[...] = _bn_fold(gacc[...], sacc[0, :], w_ref[...],
                                 gb_ref[0, :], gb_ref[1, :])


def _tc_stats2_body(smax_ref, w_ref, gb_ref, scsh_ref, gacc, sacc):
    b = pl.program_id(0)
    j = pl.program_id(1)
    x = smax_ref[...]

    @pl.when((b == 0) & (j == 0))
    def _():
        gacc[...] = jnp.zeros_like(gacc)
        sacc[...] = jnp.zeros_like(sacc)

    gacc[...] += lax.dot_general(x, x, dimension_numbers=(((0,), (0,)), ((), ())),
                                 preferred_element_type=jnp.float32)
    sacc[0, :] += jnp.sum(x, axis=0)

    @pl.when((b == B - 1) & (j == NJ - 1))
    def _():
        scsh_ref[...] = _bn_fold(gacc[...], sacc[0, :], w_ref[...],
                                 gb_ref[0, :], gb_ref[1, :])


def _tc_apply_body(sp_ref, st_ref, smax_ref, w1s_ref, w1t_ref, w2_ref,
                   prm_ref, sp_out_ref, st_out_ref):
    y1 = (jnp.dot(w1s_ref[...], sp_ref[0], preferred_element_type=jnp.float32)
          + jnp.dot(w1t_ref[...], st_ref[0], preferred_element_type=jnp.float32))
    y2 = lax.dot_general(w2_ref[...], smax_ref[...],
                         dimension_numbers=(((1,), (1,)), ((), ())),
                         preferred_element_type=jnp.float32)
    sc1 = prm_ref[0, :]
    sh1 = prm_ref[1, :]
    sc2 = prm_ref[2, :]
    sh2 = prm_ref[3, :]
    sp_out_ref[0] = jnp.maximum(y1 * sc1[:, None] + sh1[:, None], 0.0)
    st_out_ref[0] = jnp.maximum(y2 * sc2[:, None] + sh2[:, None], 0.0)


def _chan_blocks(bshape):
    return pl.BlockSpec(bshape, lambda b, j: (b, 0, j))


def _full_block(shape):
    return pl.BlockSpec(shape, lambda b, j: tuple(0 for _ in shape))


_rows_block = pl.BlockSpec((BN, C), lambda b, j: (b * NJ + j, 0))

_tc_stats1 = pl.pallas_call(
    _tc_stats1_body,
    grid=(B, NJ),
    in_specs=[
        _chan_blocks((1, C, BN)),
        _chan_blocks((1, C, BN)),
        _full_block((C, 2 * C)),
        _full_block((2, C)),
    ],
    out_specs=[_full_block((2, C))],
    out_shape=[jax.ShapeDtypeStruct((2, C), jnp.float32)],
    scratch_shapes=[
        pltpu.VMEM((2 * C, 2 * C), jnp.float32),
        pltpu.VMEM((1, 2 * C), jnp.float32),
    ],
)

_tc_stats2 = pl.pallas_call(
    _tc_stats2_body,
    grid=(B, NJ),
    in_specs=[
        _rows_block,
        _full_block((C, C)),
        _full_block((2, C)),
    ],
    out_specs=[_full_block((2, C))],
    out_shape=[jax.ShapeDtypeStruct((2, C), jnp.float32)],
    scratch_shapes=[
        pltpu.VMEM((C, C), jnp.float32),
        pltpu.VMEM((1, C), jnp.float32),
    ],
)

_tc_apply = pl.pallas_call(
    _tc_apply_body,
    grid=(B, NJ),
    in_specs=[
        _chan_blocks((1, C, BN)),
        _chan_blocks((1, C, BN)),
        _rows_block,
        _full_block((C, C)),
        _full_block((C, C)),
        _full_block((C, C)),
        _full_block((4, C)),
    ],
    out_specs=[
        _chan_blocks((1, C, BN)),
        _chan_blocks((1, C, BN)),
    ],
    out_shape=[
        jax.ShapeDtypeStruct((B, C, N), jnp.float32),
        jax.ShapeDtypeStruct((B, C, N), jnp.float32),
    ],
)


def kernel(spatial_fea, structural_fea, neighbor_index,
           W_comb, b_comb, g_comb, be_comb,
           W_agg, b_agg, g_agg, be_agg):
    table = structural_fea.transpose(0, 2, 1).reshape(M, C)
    offs = (jnp.arange(B, dtype=jnp.int32) * N)[:, None, None]
    nbr = jnp.transpose(neighbor_index + offs, (2, 0, 1)).reshape(3, M)

    smax = _sc_gather_max(table, nbr[0], nbr[1], nbr[2])

    w1s = W_comb[:, :C]
    w1t = W_comb[:, C:]
    scsh1, = _tc_stats1(spatial_fea, structural_fea, W_comb,
                        jnp.stack([g_comb, be_comb]))
    scsh2, = _tc_stats2(smax, W_agg, jnp.stack([g_agg, be_agg]))
    prm = jnp.concatenate([scsh1, scsh2], axis=0)

    sp_out, st_out = _tc_apply(spatial_fea, structural_fea, smax,
                               w1s, w1t, W_agg, prm)
    return (sp_out, st_out)
